# baseline (speedup 1.0000x reference)
import functools

import jax
import jax.numpy as jnp
from jax import lax
from jax.experimental import pallas as pl
from jax.experimental.pallas import tpu as pltpu
from jax.experimental.pallas import tpu_sc as plsc

NUM_ENTITIES = 1000000
NUM_RELATIONS = 1000
DIM = 64
BATCH = 16384

_info = plsc.get_sparse_core_info()
_NC, _NS, _L = _info.num_cores, _info.num_subcores, _info.num_lanes
_NW = _NC * _NS
_BPW = BATCH // _NW
_ICHUNK = 128
_NI = _BPW // _ICHUNK
_GROUPS = _BPW // _L

_mesh = plsc.VectorSubcoreMesh(core_axis_name="c", subcore_axis_name="s")


@functools.partial(
    pl.kernel,
    mesh=_mesh,
    out_type=jax.ShapeDtypeStruct((BATCH,), jnp.float32),
    compiler_params=pltpu.CompilerParams(
        use_tc_tiling_on_sc=False, needs_layout_passes=False),
    scratch_types=[
        pltpu.VMEM((_L * _L,), jnp.float32),
        pltpu.VMEM((_NI, _ICHUNK), jnp.int32),
        pltpu.VMEM((_NI, _ICHUNK), jnp.int32),
        pltpu.VMEM((_NI, _ICHUNK), jnp.int32),
        pltpu.VMEM((_BPW, DIM), jnp.float32),
        pltpu.VMEM((_BPW, DIM), jnp.float32),
        pltpu.VMEM((_BPW, DIM), jnp.float32),
        pltpu.VMEM((_BPW,), jnp.float32),
        pltpu.SemaphoreType.DMA,
    ],
)
def _transe_sc(h_hbm, r_hbm, t_hbm, ent_hbm, rel_hbm, out_hbm,
               part, hidx, ridx, tidx, hrows, rrows, trows, scores, sem):
    wid = lax.axis_index("s") * _NC + lax.axis_index("c")
    base = wid * _BPW

    for j in range(_NI):
        off = base + j * _ICHUNK
        pltpu.sync_copy(h_hbm.at[pl.ds(off, _ICHUNK)], hidx.at[j])
        pltpu.sync_copy(r_hbm.at[pl.ds(off, _ICHUNK)], ridx.at[j])
        pltpu.sync_copy(t_hbm.at[pl.ds(off, _ICHUNK)], tidx.at[j])
    copies = []
    for j in range(_NI):
        sl = pl.ds(j * _ICHUNK, _ICHUNK)
        copies.append(pltpu.async_copy(ent_hbm.at[hidx.at[j]], hrows.at[sl], sem))
        copies.append(pltpu.async_copy(ent_hbm.at[tidx.at[j]], trows.at[sl], sem))
        copies.append(pltpu.async_copy(rel_hbm.at[ridx.at[j]], rrows.at[sl], sem))
    for c in copies:
        c.wait()

    lane = lax.iota(jnp.int32, _L)

    def group(g, carry):
        rb = g * _L
        tot = jnp.zeros((_L,), jnp.float32)
        for i in range(_L):
            row = rb + i
            acc = jnp.zeros((_L,), jnp.float32)
            for c in range(DIM // _L):
                sl = pl.ds(c * _L, _L)
                d = hrows[row, sl] + rrows[row, sl] - trows[row, sl]
                acc = acc + d * d
            part[pl.ds(i * _L, _L)] = acc
        colbase = lane * _L
        for l in range(_L):
            tot = tot + plsc.load_gather(part, [colbase + l])
        xi = lax.bitcast_convert_type(tot, jnp.int32)
        yi = jnp.int32(0x5F3759DF) - lax.shift_right_logical(xi, 1)
        y = lax.bitcast_convert_type(yi, jnp.float32)
        for _ in range(3):
            y = y * (1.5 - 0.5 * tot * y * y)
        s = jnp.where(tot > 0.0, tot * y, 0.0)
        scores[pl.ds(rb, _L)] = s
        return carry

    lax.fori_loop(0, _GROUPS, group, 0)
    pltpu.sync_copy(scores, out_hbm.at[pl.ds(base, _BPW)])


def kernel(h, r, t, entity_embed, relation_embed):
    return _transe_sc(h, r, t, entity_embed, relation_embed)


# speedup vs baseline: 1.4688x; 1.4688x over previous
import functools

import jax
import jax.numpy as jnp
from jax import lax
from jax.experimental import pallas as pl
from jax.experimental.pallas import tpu as pltpu
from jax.experimental.pallas import tpu_sc as plsc

NUM_ENTITIES = 1000000
NUM_RELATIONS = 1000
DIM = 64
BATCH = 16384

_info = plsc.get_sparse_core_info()
_NC, _NS, _L = _info.num_cores, _info.num_subcores, _info.num_lanes
_NW = _NC * _NS
_BPW = BATCH // _NW
_BLK = _L
_NBLK = _BPW // _BLK

_mesh = plsc.VectorSubcoreMesh(core_axis_name="c", subcore_axis_name="s")


@functools.partial(
    pl.kernel,
    mesh=_mesh,
    out_type=jax.ShapeDtypeStruct((BATCH // DIM, DIM), jnp.float32),
    scratch_types=[
        pltpu.VMEM((_BPW,), jnp.int32),
        pltpu.VMEM((_BPW,), jnp.int32),
        pltpu.VMEM((_BPW,), jnp.int32),
        pltpu.VMEM((2, _BLK * 8, DIM), jnp.float32),
        pltpu.VMEM((2, _BLK * 8, DIM), jnp.float32),
        pltpu.VMEM((2, _BLK * 8, DIM), jnp.float32),
        pltpu.VMEM((_BPW // DIM, DIM), jnp.float32),
        pltpu.SemaphoreType.DMA,
    ],
)
def _transe_sc(h_hbm, r_hbm, t_hbm, ent_hbm, rel_hbm, out_hbm,
               hidx, ridx, tidx, hblk, rblk, tblk, scores, sem):
    wid = lax.axis_index("s") * _NC + lax.axis_index("c")
    base = wid * _BPW

    pltpu.sync_copy(h_hbm.at[pl.ds(base, _BPW)], hidx)
    pltpu.sync_copy(r_hbm.at[pl.ds(base, _BPW)], ridx)
    pltpu.sync_copy(t_hbm.at[pl.ds(base, _BPW)], tidx)

    lane = lax.iota(jnp.int32, _L)

    def fire(b, q):
        hv = hidx[pl.ds(b * _BLK, _BLK)]
        tv = tidx[pl.ds(b * _BLK, _BLK)]
        rv = ridx[pl.ds(b * _BLK, _BLK)]
        for i in range(_BLK):
            hb = pl.multiple_of(lax.shift_right_logical(hv[i], 3) * 8, 8)
            tb = pl.multiple_of(lax.shift_right_logical(tv[i], 3) * 8, 8)
            rb = pl.multiple_of(lax.shift_right_logical(rv[i], 3) * 8, 8)
            dsl = pl.ds(i * 8, 8)
            pltpu.async_copy(ent_hbm.at[pl.ds(hb, 8)], hblk.at[q, dsl], sem)
            pltpu.async_copy(ent_hbm.at[pl.ds(tb, 8)], tblk.at[q, dsl], sem)
            pltpu.async_copy(rel_hbm.at[pl.ds(rb, 8)], rblk.at[q, dsl], sem)

    def drain(q):
        src = ent_hbm.at[pl.ds(0, 8)]
        for i in range(_BLK):
            dsl = pl.ds(i * 8, 8)
            pltpu.make_async_copy(src, hblk.at[q, dsl], sem).wait()
            pltpu.make_async_copy(src, tblk.at[q, dsl], sem).wait()
            pltpu.make_async_copy(src, rblk.at[q, dsl], sem).wait()

    def process(b, q):
        hv = hidx[pl.ds(b * _BLK, _BLK)]
        tv = tidx[pl.ds(b * _BLK, _BLK)]
        rv = ridx[pl.ds(b * _BLK, _BLK)]
        tot = jnp.zeros((_L,), jnp.float32)
        for i in range(_BLK):
            hrow = i * 8 + (hv[i] & 7)
            trow = i * 8 + (tv[i] & 7)
            rrow = i * 8 + (rv[i] & 7)
            acc = jnp.zeros((_L,), jnp.float32)
            for c in range(DIM // _L):
                sl = pl.ds(c * _L, _L)
                d = hblk[q, hrow, sl] + rblk[q, rrow, sl] - tblk[q, trow, sl]
                acc = acc + d * d
            for k in (8, 4, 2, 1):
                acc = acc + acc.at[lane ^ k].get(mode="promise_in_bounds")
            tot = jnp.where(lane == i, acc, tot)
        xi = lax.bitcast_convert_type(tot, jnp.int32)
        yi = jnp.int32(0x5F3759DF) - lax.shift_right_logical(xi, 1)
        y = lax.bitcast_convert_type(yi, jnp.float32)
        for _ in range(3):
            y = y * (1.5 - 0.5 * tot * y * y)
        s = jnp.where(tot > 0.0, tot * y, 0.0)
        scores[lax.shift_right_logical(b, 2), pl.ds((b & 3) * _L, _L)] = s

    fire(0, 0)

    def pipe(b, carry):
        q = b & 1

        @pl.when(b + 1 < _NBLK)
        def _():
            fire(b + 1, q ^ 1)

        drain(q)
        process(b, q)
        return carry

    lax.fori_loop(0, _NBLK, pipe, 0)
    pltpu.sync_copy(scores, out_hbm.at[pl.ds(wid * 8, 8)])


def kernel(h, r, t, entity_embed, relation_embed):
    out2 = _transe_sc(h, r, t, entity_embed, relation_embed)
    return out2.reshape(BATCH)


# speedup vs baseline: 1.4691x; 1.0002x over previous
import functools

import jax
import jax.numpy as jnp
from jax import lax
from jax.experimental import pallas as pl
from jax.experimental.pallas import tpu as pltpu
from jax.experimental.pallas import tpu_sc as plsc

NUM_ENTITIES = 1000000
NUM_RELATIONS = 1000
DIM = 64
BATCH = 16384

_info = plsc.get_sparse_core_info()
_NC, _NS, _L = _info.num_cores, _info.num_subcores, _info.num_lanes
_NW = _NC * _NS
_BPW = BATCH // _NW
_BLK = _L
_NBLK = _BPW // _BLK

_mesh = plsc.VectorSubcoreMesh(core_axis_name="c", subcore_axis_name="s")


@functools.partial(
    pl.kernel,
    mesh=_mesh,
    out_type=jax.ShapeDtypeStruct((BATCH // DIM, DIM), jnp.float32),
    scratch_types=[
        pltpu.VMEM((_BPW,), jnp.int32),
        pltpu.VMEM((_BPW,), jnp.int32),
        pltpu.VMEM((_BPW,), jnp.int32),
        pltpu.VMEM((2, _BLK * 8, DIM), jnp.float32),
        pltpu.VMEM((2, _BLK * 8, DIM), jnp.float32),
        pltpu.VMEM((2, _BLK * 8, DIM), jnp.float32),
        pltpu.VMEM((_BPW // DIM, DIM), jnp.float32),
        pltpu.SemaphoreType.DMA,
    ],
)
def _transe_sc(h_hbm, r_hbm, t_hbm, ent_hbm, rel_hbm, out_hbm,
               hidx, ridx, tidx, hblk, tblk, rblk, scores, sem):
    wid = lax.axis_index("s") * _NC + lax.axis_index("c")
    base = wid * _BPW

    pltpu.sync_copy(h_hbm.at[pl.ds(base, _BPW)], hidx)
    pltpu.sync_copy(r_hbm.at[pl.ds(base, _BPW)], ridx)
    pltpu.sync_copy(t_hbm.at[pl.ds(base, _BPW)], tidx)

    lane = lax.iota(jnp.int32, _L)

    def fire(b, q):
        hv = hidx[pl.ds(b * _BLK, _BLK)]
        tv = tidx[pl.ds(b * _BLK, _BLK)]
        rv = ridx[pl.ds(b * _BLK, _BLK)]
        for i in range(_BLK):
            hb = pl.multiple_of(lax.shift_right_logical(hv[i], 3) * 8, 8)
            tb = pl.multiple_of(lax.shift_right_logical(tv[i], 3) * 8, 8)
            rb = pl.multiple_of(lax.shift_right_logical(rv[i], 3) * 8, 8)
            dsl = pl.ds(i * 8, 8)
            pltpu.async_copy(ent_hbm.at[pl.ds(hb, 8)], hblk.at[q, dsl], sem)
            pltpu.async_copy(ent_hbm.at[pl.ds(tb, 8)], tblk.at[q, dsl], sem)
            pltpu.async_copy(rel_hbm.at[pl.ds(rb, 8)], rblk.at[q, dsl], sem)

    def drain(q):
        src = ent_hbm.at[pl.ds(0, _BLK * 8)]
        pltpu.make_async_copy(src, hblk.at[q], sem).wait()
        pltpu.make_async_copy(src, tblk.at[q], sem).wait()
        pltpu.make_async_copy(src, rblk.at[q], sem).wait()

    def process(b, q):
        hv = hidx[pl.ds(b * _BLK, _BLK)]
        tv = tidx[pl.ds(b * _BLK, _BLK)]
        rv = ridx[pl.ds(b * _BLK, _BLK)]
        tot = jnp.zeros((_L,), jnp.float32)
        for i in range(_BLK):
            hrow = i * 8 + (hv[i] & 7)
            trow = i * 8 + (tv[i] & 7)
            rrow = i * 8 + (rv[i] & 7)
            acc = jnp.zeros((_L,), jnp.float32)
            for c in range(DIM // _L):
                sl = pl.ds(c * _L, _L)
                d = hblk[q, hrow, sl] + rblk[q, rrow, sl] - tblk[q, trow, sl]
                acc = acc + d * d
            for k in (8, 4, 2, 1):
                acc = acc + acc.at[lane ^ k].get(mode="promise_in_bounds")
            tot = jnp.where(lane == i, acc, tot)
        xi = lax.bitcast_convert_type(tot, jnp.int32)
        yi = jnp.int32(0x5F3759DF) - lax.shift_right_logical(xi, 1)
        y = lax.bitcast_convert_type(yi, jnp.float32)
        for _ in range(3):
            y = y * (1.5 - 0.5 * tot * y * y)
        s = jnp.where(tot > 0.0, tot * y, 0.0)
        scores[lax.shift_right_logical(b, 2), pl.ds((b & 3) * _L, _L)] = s

    fire(0, 0)

    def pipe(b, carry):
        q = b & 1

        @pl.when(b + 1 < _NBLK)
        def _():
            fire(b + 1, q ^ 1)

        drain(q)
        process(b, q)
        return carry

    lax.fori_loop(0, _NBLK, pipe, 0)
    pltpu.sync_copy(scores, out_hbm.at[pl.ds(wid * 8, 8)])


def kernel(h, r, t, entity_embed, relation_embed):
    out2 = _transe_sc(h, r, t, entity_embed, relation_embed)
    return out2.reshape(BATCH)
